# Initial kernel scaffold; baseline (speedup 1.0000x reference)
import jax
import jax.numpy as jnp
from jax.experimental import pallas as pl


def kernel(numbers, table):
    raise NotImplementedError("write your pallas kernel here")


# speedup vs baseline: 3.7252x; 3.7252x over previous
import functools

import jax
import jax.numpy as jnp
from jax import lax
from jax.experimental import pallas as pl
from jax.experimental.pallas import tpu as pltpu
from jax.experimental.pallas import tpu_sc as plsc

_NC, _NS = 2, 16
_NW = _NC * _NS
_CA = 128


def _rank_body(x_ref, out_a_ref, out_b_ref, *, n, bb):
    x = x_ref[...]
    xi = x[:, :, None]
    xj = x[:, None, :]
    lt = xj < xi
    le = xj <= xi
    ii = lax.broadcasted_iota(jnp.int32, (bb, n, n), 1)
    jj = lax.broadcasted_iota(jnp.int32, (bb, n, n), 2)
    c = jnp.where(lt | ((jj < ii) & le), 1.0, 0.0)
    rank = jnp.sum(c, axis=-1).astype(jnp.int32)
    row = pl.program_id(0) * bb + lax.broadcasted_iota(jnp.int32, (bb, n), 0)
    offs = rank + row * n
    out_a_ref[...] = offs[:, :_CA]
    out_b_ref[...] = offs[:, _CA:]


def _rank_call(numbers, bb=8, interpret=False):
    b, n = numbers.shape
    return pl.pallas_call(
        functools.partial(_rank_body, n=n, bb=bb),
        grid=(b // bb,),
        in_specs=[pl.BlockSpec((bb, n), lambda i: (i, 0))],
        out_specs=[
            pl.BlockSpec((bb, _CA), lambda i: (i, 0)),
            pl.BlockSpec((bb, n - _CA), lambda i: (i, 0)),
        ],
        out_shape=[
            jax.ShapeDtypeStruct((b, _CA), jnp.int32),
            jax.ShapeDtypeStruct((b, n - _CA), jnp.int32),
        ],
        interpret=interpret,
    )(numbers)


def _make_sc_scatter(b, n, d):
    rpw = b // _NW
    cb = n - _CA
    mesh = plsc.VectorSubcoreMesh(
        core_axis_name="c", subcore_axis_name="s",
        num_cores=_NC, num_subcores=_NS,
    )

    @functools.partial(
        pl.kernel,
        out_type=jax.ShapeDtypeStruct((b * n, d), jnp.float32),
        mesh=mesh,
        compiler_params=pltpu.CompilerParams(use_tc_tiling_on_sc=False),
        scratch_types=[
            pltpu.VMEM((n, d), jnp.float32),
            pltpu.VMEM((rpw, _CA), jnp.int32),
            pltpu.VMEM((rpw, cb), jnp.int32),
            pltpu.SemaphoreType.DMA,
        ],
    )
    def sc_scatter(offs_a, offs_b, table_hbm, out_hbm, tbl_v, ia_v, ib_v, sem):
        wid = lax.axis_index("s") * _NC + lax.axis_index("c")
        pltpu.sync_copy(table_hbm.at[pl.ds(0, n)], tbl_v)
        pltpu.sync_copy(offs_a.at[pl.ds(wid * rpw, rpw)], ia_v)
        pltpu.sync_copy(offs_b.at[pl.ds(wid * rpw, rpw)], ib_v)

        def drain_pair():
            pltpu.make_async_copy(
                table_hbm.at[pl.ds(0, _CA)], tbl_v.at[pl.ds(0, _CA)], sem
            ).wait()
            pltpu.make_async_copy(
                table_hbm.at[pl.ds(0, cb)], tbl_v.at[pl.ds(_CA, cb)], sem
            ).wait()

        def body(bi, carry):
            pltpu.async_copy(
                tbl_v.at[pl.ds(0, _CA)], out_hbm.at[ia_v.at[bi]], sem
            )
            pltpu.async_copy(
                tbl_v.at[pl.ds(_CA, cb)], out_hbm.at[ib_v.at[bi]], sem
            )
            pl.when(bi >= 1)(drain_pair)
            return carry

        lax.fori_loop(0, rpw, body, 0)
        drain_pair()

    return sc_scatter


def kernel(numbers, table):
    b, n = numbers.shape
    _, d = table.shape
    offs_a, offs_b = _rank_call(numbers)
    out = _make_sc_scatter(b, n, d)(offs_a, offs_b, table)
    return out.reshape(b, n, d)
